# baseline (speedup 1.0000x reference)
import functools

import jax
import jax.numpy as jnp
from jax import lax
from jax.experimental import pallas as pl
from jax.experimental.pallas import tpu as pltpu
from jax.experimental.pallas import tpu_sc as plsc

N, D, H, G = 100000, 128, 128, 64
B = 20000
NB = N // B
NEG = -1e30

SC_W = 32
C = 3136
CL = N - (SC_W - 1) * C


def _pool_body(x_ref, seg_ref, w1_ref, b1_ref, w2_ref, b2_ref,
               logits_ref, m_ref, d_ref, pooled_ref, t_ref):
    i = pl.program_id(0)

    @pl.when(i == 0)
    def _init():
        m_ref[...] = jnp.full((G, 1), NEG, jnp.float32)
        d_ref[...] = jnp.zeros((G, 1), jnp.float32)
        pooled_ref[...] = jnp.zeros((G, D), jnp.float32)

    x_b = x_ref[...]
    h = jnp.tanh(jnp.dot(x_b, w1_ref[...],
                         preferred_element_type=jnp.float32) + b1_ref[...])
    lg = lax.dot_general(w2_ref[...], h, (((0,), (1,)), ((), ())),
                         preferred_element_type=jnp.float32) + b2_ref[...]
    logits_ref[0] = lg

    seg = seg_ref[0]
    oh = lax.broadcasted_iota(jnp.int32, (G, B), 0) == seg
    ohf = oh.astype(jnp.float32)

    bm = jnp.max(jnp.where(oh, jnp.broadcast_to(lg, (G, B)), NEG),
                 axis=1, keepdims=True)
    m_old = m_ref[...]
    m_new = jnp.maximum(m_old, bm)
    scale = jnp.exp(m_old - m_new)

    m_g = lax.dot_general(m_new, ohf, (((0,), (0,)), ((), ())),
                          preferred_element_type=jnp.float32)
    e = jnp.exp(lg - m_g)
    we = ohf * e

    d_ref[...] = d_ref[...] * scale + jnp.sum(we, axis=1, keepdims=True)
    pooled_ref[...] = (pooled_ref[...] * scale
                       + jnp.dot(we, x_b,
                                 preferred_element_type=jnp.float32))
    m_ref[...] = m_new

    @pl.when(i == NB - 1)
    def _fin():
        d_c = d_ref[...]
        pooled_ref[...] = jnp.where(d_c > 0, pooled_ref[...] / d_c, 0.0)
        t_ref[...] = m_ref[...] + jnp.where(d_c > 0, jnp.log(d_c), 0.0)


_pool_call = pl.pallas_call(
    _pool_body,
    grid=(NB,),
    in_specs=[
        pl.BlockSpec((B, D), lambda i: (i, 0)),
        pl.BlockSpec((1, 1, B), lambda i: (i, 0, 0)),
        pl.BlockSpec((D, H), lambda i: (0, 0)),
        pl.BlockSpec((1, H), lambda i: (0, 0)),
        pl.BlockSpec((H, 1), lambda i: (0, 0)),
        pl.BlockSpec((1, 1), lambda i: (0, 0)),
    ],
    out_specs=[
        pl.BlockSpec((1, 1, B), lambda i: (i, 0, 0)),
        pl.BlockSpec((G, 1), lambda i: (0, 0)),
        pl.BlockSpec((G, 1), lambda i: (0, 0)),
        pl.BlockSpec((G, D), lambda i: (0, 0)),
        pl.BlockSpec((G, 1), lambda i: (0, 0)),
    ],
    out_shape=[
        jax.ShapeDtypeStruct((NB, 1, B), jnp.float32),
        jax.ShapeDtypeStruct((G, 1), jnp.float32),
        jax.ShapeDtypeStruct((G, 1), jnp.float32),
        jax.ShapeDtypeStruct((G, D), jnp.float32),
        jax.ShapeDtypeStruct((G, 1), jnp.float32),
    ],
)


@functools.cache
def _sc_gate_kernel():
    @functools.partial(
        pl.kernel,
        mesh=plsc.VectorSubcoreMesh(core_axis_name="c", subcore_axis_name="s"),
        out_type=jax.ShapeDtypeStruct((N,), jnp.float32),
        scratch_types=[
            pltpu.VMEM((C,), jnp.float32),
            pltpu.VMEM((C,), jnp.int32),
            pltpu.VMEM((G,), jnp.float32),
            pltpu.VMEM((C,), jnp.float32),
        ],
    )
    def _sc_gate(lg_hbm, seg_hbm, t_hbm, out_hbm,
                 lg_v, seg_v, t_v, o_v):
        wid = lax.axis_index("s") * 2 + lax.axis_index("c")
        base = wid * C
        pltpu.sync_copy(t_hbm, t_v)

        tt = [t_v[pl.ds(k * 16, 16)] for k in range(G // 16)]

        def lut(tabs, hi, lo):
            out = tabs[0].at[lo].get(mode="promise_in_bounds")
            for k in range(1, G // 16):
                out = jnp.where(hi == k,
                                tabs[k].at[lo].get(mode="promise_in_bounds"),
                                out)
            return out

        def run(count):
            pltpu.sync_copy(lg_hbm.at[pl.ds(base, count)],
                            lg_v.at[pl.ds(0, count)])
            pltpu.sync_copy(seg_hbm.at[pl.ds(base, count)],
                            seg_v.at[pl.ds(0, count)])

            def body(j, carry):
                sl = pl.ds(j * 16, 16)
                seg = seg_v[sl]
                hi = seg >> 4
                lo = seg & 15
                o_v[sl] = jnp.exp(lg_v[sl] - lut(tt, hi, lo))
                return carry

            lax.fori_loop(0, count // 16, body, 0, unroll=2)
            pltpu.sync_copy(o_v.at[pl.ds(0, count)],
                            out_hbm.at[pl.ds(base, count)])

        @pl.when(wid < SC_W - 1)
        def _full():
            run(C)

        @pl.when(wid == SC_W - 1)
        def _tail():
            run(CL)

    return _sc_gate


def kernel(x, batch, W1, b1, W2, b2):
    seg = batch.astype(jnp.int32)
    logits3, m, d, pooled, t = _pool_call(
        x, seg.reshape(NB, 1, B), W1, b1.reshape(1, H), W2, b2.reshape(1, 1))
    gate = _sc_gate_kernel()(logits3.reshape(N), seg, t.reshape(G))
    return (pooled, gate)


# speedup vs baseline: 1.0390x; 1.0390x over previous
import functools

import jax
import jax.numpy as jnp
from jax import lax
from jax.experimental import pallas as pl
from jax.experimental.pallas import tpu as pltpu
from jax.experimental.pallas import tpu_sc as plsc

N, D, H, G = 100000, 128, 128, 64
B = 20000
NB = N // B
NEG = -1e30

SC_W = 32
C = 3136
CL = N - (SC_W - 1) * C


def _pool_body(x_ref, seg_ref, w1_ref, b1_ref, w2_ref, b2_ref,
               logits_ref, m_ref, d_ref, pooled_ref, t_ref):
    i = pl.program_id(0)

    @pl.when(i == 0)
    def _init():
        m_ref[...] = jnp.full((G, 1), NEG, jnp.float32)
        d_ref[...] = jnp.zeros((G, 1), jnp.float32)
        pooled_ref[...] = jnp.zeros((G, D), jnp.float32)

    x_b = x_ref[...]
    h = jnp.tanh(jnp.dot(x_b, w1_ref[...],
                         preferred_element_type=jnp.float32) + b1_ref[...])
    lg = lax.dot_general(w2_ref[...], h, (((0,), (1,)), ((), ())),
                         preferred_element_type=jnp.float32) + b2_ref[...]
    logits_ref[0] = lg

    seg = seg_ref[0]
    oh = lax.broadcasted_iota(jnp.int32, (G, B), 0) == seg
    ohf = oh.astype(jnp.float32)

    bm = jnp.max(jnp.where(oh, jnp.broadcast_to(lg, (G, B)), NEG),
                 axis=1, keepdims=True)
    m_old = m_ref[...]
    m_new = jnp.maximum(m_old, bm)
    scale = jnp.exp(m_old - m_new)

    m_g = lax.dot_general(m_new, ohf, (((0,), (0,)), ((), ())),
                          preferred_element_type=jnp.float32)
    e = jnp.exp(lg - m_g)
    we = ohf * e

    d_ref[...] = d_ref[...] * scale + jnp.sum(we, axis=1, keepdims=True)
    pooled_ref[...] = (pooled_ref[...] * scale
                       + jnp.dot(we, x_b,
                                 preferred_element_type=jnp.float32))
    m_ref[...] = m_new

    @pl.when(i == NB - 1)
    def _fin():
        d_c = d_ref[...]
        pooled_ref[...] = jnp.where(d_c > 0, pooled_ref[...] / d_c, 0.0)
        t_ref[...] = m_ref[...] + jnp.where(d_c > 0, jnp.log(d_c), 0.0)


_pool_call = pl.pallas_call(
    _pool_body,
    grid=(NB,),
    in_specs=[
        pl.BlockSpec((B, D), lambda i: (i, 0)),
        pl.BlockSpec((1, 1, B), lambda i: (i, 0, 0)),
        pl.BlockSpec((D, H), lambda i: (0, 0)),
        pl.BlockSpec((1, H), lambda i: (0, 0)),
        pl.BlockSpec((H, 1), lambda i: (0, 0)),
        pl.BlockSpec((1, 1), lambda i: (0, 0)),
    ],
    out_specs=[
        pl.BlockSpec((1, 1, B), lambda i: (i, 0, 0)),
        pl.BlockSpec((G, 1), lambda i: (0, 0)),
        pl.BlockSpec((G, 1), lambda i: (0, 0)),
        pl.BlockSpec((G, D), lambda i: (0, 0)),
        pl.BlockSpec((G, 1), lambda i: (0, 0)),
    ],
    out_shape=[
        jax.ShapeDtypeStruct((NB, 1, B), jnp.float32),
        jax.ShapeDtypeStruct((G, 1), jnp.float32),
        jax.ShapeDtypeStruct((G, 1), jnp.float32),
        jax.ShapeDtypeStruct((G, D), jnp.float32),
        jax.ShapeDtypeStruct((G, 1), jnp.float32),
    ],
)


@functools.cache
def _sc_gate_kernel():
    @functools.partial(
        pl.kernel,
        mesh=plsc.VectorSubcoreMesh(core_axis_name="c", subcore_axis_name="s"),
        out_type=jax.ShapeDtypeStruct((N,), jnp.float32),
        scratch_types=[
            pltpu.VMEM((C,), jnp.float32),
            pltpu.VMEM((C,), jnp.int32),
            pltpu.VMEM((G,), jnp.float32),
            pltpu.VMEM((C,), jnp.float32),
        ],
    )
    def _sc_gate(lg_hbm, seg_hbm, t_hbm, out_hbm,
                 lg_v, seg_v, t_v, o_v):
        wid = lax.axis_index("s") * 2 + lax.axis_index("c")
        base = wid * C
        pltpu.sync_copy(t_hbm, t_v)

        tt = [t_v[pl.ds(k * 16, 16)] for k in range(G // 16)]

        def lut(tabs, hi, lo):
            out = tabs[0].at[lo].get(mode="promise_in_bounds")
            for k in range(1, G // 16):
                out = jnp.where(hi == k,
                                tabs[k].at[lo].get(mode="promise_in_bounds"),
                                out)
            return out

        def run(count):
            pltpu.sync_copy(lg_hbm.at[pl.ds(base, count)],
                            lg_v.at[pl.ds(0, count)])
            pltpu.sync_copy(seg_hbm.at[pl.ds(base, count)],
                            seg_v.at[pl.ds(0, count)])

            def body(j, carry):
                sl = pl.ds(j * 16, 16)
                seg = seg_v[sl]
                hi = seg >> 4
                lo = seg & 15
                o_v[sl] = jnp.exp(lg_v[sl] - lut(tt, hi, lo))
                return carry

            lax.fori_loop(0, count // 16, body, 0)
            pltpu.sync_copy(o_v.at[pl.ds(0, count)],
                            out_hbm.at[pl.ds(base, count)])

        @pl.when(wid < SC_W - 1)
        def _full():
            run(C)

        @pl.when(wid == SC_W - 1)
        def _tail():
            run(CL)

    return _sc_gate


def kernel(x, batch, W1, b1, W2, b2):
    seg = batch.astype(jnp.int32)
    logits3, m, d, pooled, t = _pool_call(
        x, seg.reshape(NB, 1, B), W1, b1.reshape(1, H), W2, b2.reshape(1, 1))
    gate = _sc_gate_kernel()(logits3.reshape(N), seg, t.reshape(G))
    return (pooled, gate)
